# baseline (speedup 1.0000x reference)
import jax
import jax.numpy as jnp
from jax.experimental import pallas as pl

TEMP = 50.0
BN = 2048


def _vq_kernel(xt_ref, c_ref, o_ref):
    xt = xt_ref[:]
    c = c_ref[:]
    cnorm = jnp.sum(c * c, axis=1, keepdims=True)
    logits = (-TEMP) * cnorm + (2.0 * TEMP) * c[:, 0:1] * xt[0:1, :]
    for d in range(1, 4):
        logits = logits + (2.0 * TEMP) * c[:, d : d + 1] * xt[d : d + 1, :]
    m = jnp.max(logits, axis=0, keepdims=True)
    e = jnp.exp(logits - m)
    caug = jnp.concatenate(
        [c, jnp.ones((c.shape[0], 1), jnp.float32)], axis=1
    )
    w = jax.lax.dot_general(
        caug,
        e,
        (((0,), (0,)), ((), ())),
        preferred_element_type=jnp.float32,
    )
    o_ref[:] = w[0:4, :] / w[4:5, :]


def kernel(x, center):
    B, F = x.shape
    n = (B * F) // 4
    xt = x.reshape(n, 4).T
    grid = n // BN
    ot = pl.pallas_call(
        _vq_kernel,
        grid=(grid,),
        in_specs=[
            pl.BlockSpec((4, BN), lambda i: (0, i)),
            pl.BlockSpec((512, 4), lambda i: (0, 0)),
        ],
        out_specs=pl.BlockSpec((4, BN), lambda i: (0, i)),
        out_shape=jax.ShapeDtypeStruct((4, n), jnp.float32),
    )(xt, center)
    return ot.T.reshape(B, F)
